# Initial kernel scaffold; baseline (speedup 1.0000x reference)
import jax
import jax.numpy as jnp
from jax.experimental import pallas as pl


def kernel(x, weight):
    raise NotImplementedError("write your pallas kernel here")


# speedup vs baseline: 1.3873x; 1.3873x over previous
import jax
import jax.numpy as jnp
from jax.experimental import pallas as pl
from jax.experimental.pallas import tpu as pltpu

_DIM = 2048
_N_EXPERTS = 64
_TOPK = 8
_BLOCK = 512


def _gate_block_kernel(x_ref, wt_ref, w_out_ref, i_out_ref):
    x = x_ref[...]
    wt = wt_ref[...]
    scores = jax.lax.dot_general(
        x, wt, dimension_numbers=(((1,), (0,)), ((), ())),
        preferred_element_type=jnp.float32)
    m = jnp.max(scores, axis=-1, keepdims=True)
    e = jnp.exp(scores - m)
    probs = e / jnp.sum(e, axis=-1, keepdims=True)

    expert_ids = jax.lax.broadcasted_iota(jnp.int32, probs.shape, 1)
    s = probs
    ws = []
    ids = []
    for _ in range(_TOPK):
        mk = jnp.max(s, axis=-1, keepdims=True)
        is_max = s == mk
        idx = jnp.min(jnp.where(is_max, expert_ids, _N_EXPERTS),
                      axis=-1, keepdims=True)
        ws.append(mk)
        ids.append(idx)
        s = jnp.where(expert_ids == idx, -1.0, s)
    w_out_ref[...] = jnp.concatenate(ws, axis=-1)
    i_out_ref[...] = jnp.concatenate(ids, axis=-1)


def kernel(x, weight):
    tokens = x.shape[0]
    wt = weight.T
    grid = (tokens // _BLOCK,)
    w_out, i_out = pl.pallas_call(
        _gate_block_kernel,
        grid=grid,
        in_specs=[
            pl.BlockSpec((_BLOCK, _DIM), lambda i: (i, 0)),
            pl.BlockSpec((_DIM, _N_EXPERTS), lambda i: (0, 0)),
        ],
        out_specs=[
            pl.BlockSpec((_BLOCK, _TOPK), lambda i: (i, 0)),
            pl.BlockSpec((_BLOCK, _TOPK), lambda i: (i, 0)),
        ],
        out_shape=[
            jax.ShapeDtypeStruct((tokens, _TOPK), jnp.float32),
            jax.ShapeDtypeStruct((tokens, _TOPK), jnp.int32),
        ],
    )(x, wt)
    return (w_out, i_out)
